# baseline (speedup 1.0000x reference)
import functools

import jax
import jax.numpy as jnp
from jax import lax
from jax.experimental import pallas as pl
from jax.experimental.pallas import tpu as pltpu
from jax.experimental.pallas import tpu_sc as plsc

N = 10000
NP = 10240
E = 320000
D = 128
DH = D // 2
NC = 2
NS = 16
CH = 80
ROWS_PER_TILE = NP // NS
ACH = 128
AGG_EDGES_PER_TILE = E // NS
AGG_CHUNKS = 160
AGG_EDGES_PAD = AGG_CHUNKS * ACH
PAD_NODE = NP - 8
DEG_EDGES_PER_TILE = E // (NC * NS)
DEG_CHUNKS = DEG_EDGES_PER_TILE // CH
DEG_W = 16
BLK = 1024

_MESH = plsc.VectorSubcoreMesh(core_axis_name="c", subcore_axis_name="s")


@functools.partial(
    pl.kernel,
    out_type=jax.ShapeDtypeStruct((NC * NP, DEG_W), jnp.float32),
    mesh=_MESH,
    compiler_params=pltpu.CompilerParams(use_tc_tiling_on_sc=False),
    scratch_types=[
        pltpu.VMEM((DEG_CHUNKS, CH), jnp.int32),
        pltpu.VMEM((CH, DEG_W), jnp.float32),
        pltpu.VMEM((ROWS_PER_TILE, DEG_W), jnp.float32),
        pltpu.VMEM_SHARED((NP, DEG_W), jnp.float32),
    ],
)
def _deg_kernel(dst_hbm, out_hbm, idx_v, ones_v, zeros_v, acc_sh):
    c = lax.axis_index("c")
    s = lax.axis_index("s")
    wid = c * NS + s
    one_row = jnp.ones((16,), jnp.float32)
    zero_row = jnp.zeros((16,), jnp.float32)

    @pl.loop(0, CH)
    def _(i):
        ones_v[i] = one_row

    @pl.loop(0, ROWS_PER_TILE)
    def _(i):
        zeros_v[i] = zero_row

    pltpu.sync_copy(zeros_v, acc_sh.at[pl.ds(s * ROWS_PER_TILE, ROWS_PER_TILE)])
    pltpu.sync_copy(dst_hbm.at[wid], idx_v)
    plsc.subcore_barrier()

    @pl.loop(0, DEG_CHUNKS)
    def _(j):
        pltpu.sync_copy(ones_v, acc_sh.at[idx_v.at[j]], add=True)

    plsc.subcore_barrier()
    pltpu.sync_copy(acc_sh.at[pl.ds(s * ROWS_PER_TILE, ROWS_PER_TILE)],
                    out_hbm.at[pl.ds(c * NP + s * ROWS_PER_TILE, ROWS_PER_TILE)])


@functools.partial(
    pl.kernel,
    out_type=jax.ShapeDtypeStruct((NC * NP, DH), jnp.float32),
    mesh=_MESH,
    compiler_params=pltpu.CompilerParams(use_tc_tiling_on_sc=False),
    scratch_types=[
        pltpu.VMEM((AGG_CHUNKS, ACH), jnp.int32),
        pltpu.VMEM((AGG_CHUNKS, ACH), jnp.int32),
        pltpu.VMEM((ACH, DH), jnp.float32),
        pltpu.VMEM((ACH, DH), jnp.float32),
        pltpu.VMEM_SHARED((NP, DH), jnp.float32),
        pltpu.SemaphoreType.DMA,
        pltpu.SemaphoreType.DMA,
    ],
)
def _agg_kernel(g_hbm, src2_hbm, dst_hbm, out_hbm,
                src_v, dst_v, rows0_v, rows1_v, acc_sh, sem0, sem1):
    c = lax.axis_index("c")
    s = lax.axis_index("s")
    base = c * NP + s * ROWS_PER_TILE
    pltpu.sync_copy(g_hbm.at[pl.ds(base, ROWS_PER_TILE)],
                    acc_sh.at[pl.ds(s * ROWS_PER_TILE, ROWS_PER_TILE)])
    pltpu.sync_copy(src2_hbm.at[c, s], src_v)
    pltpu.sync_copy(dst_hbm.at[s], dst_v)
    plsc.subcore_barrier()

    pltpu.async_copy(g_hbm.at[src_v.at[0]], rows0_v, sem0)
    pltpu.async_copy(g_hbm.at[src_v.at[1]], rows1_v, sem1)

    @pl.loop(0, AGG_CHUNKS, step=2)
    def _(j):
        pltpu.make_async_copy(g_hbm.at[src_v.at[j]], rows0_v, sem0).wait()
        pltpu.sync_copy(rows0_v, acc_sh.at[dst_v.at[j]], add=True)

        @pl.when(j + 2 < AGG_CHUNKS)
        def _():
            pltpu.async_copy(g_hbm.at[src_v.at[j + 2]], rows0_v, sem0)

        pltpu.make_async_copy(g_hbm.at[src_v.at[j + 1]], rows1_v, sem1).wait()
        pltpu.sync_copy(rows1_v, acc_sh.at[dst_v.at[j + 1]], add=True)

        @pl.when(j + 3 < AGG_CHUNKS)
        def _():
            pltpu.async_copy(g_hbm.at[src_v.at[j + 3]], rows1_v, sem1)

    plsc.subcore_barrier()
    pltpu.sync_copy(acc_sh.at[pl.ds(s * ROWS_PER_TILE, ROWS_PER_TILE)],
                    out_hbm.at[pl.ds(base, ROWS_PER_TILE)])


def _dinv_of(pd_ref):
    deg = pd_ref[0, :, 0] + pd_ref[1, :, 0] + 1.0
    return lax.rsqrt(deg)


def _mm1_body(pd_ref, x_ref, w_ref, g_ref):
    dinv = _dinv_of(pd_ref)
    h = jnp.dot(x_ref[...], w_ref[...], preferred_element_type=jnp.float32)
    g = h * dinv[:, None]
    g_ref[0] = g[:, :DH]
    g_ref[1] = g[:, DH:]


def _mm2_body(pd_ref, a_ref, b_ref, w_ref, g_ref):
    dinv = _dinv_of(pd_ref)
    a = jnp.concatenate([a_ref[0], a_ref[1]], axis=1)
    h = jnp.maximum(a * dinv[:, None] + b_ref[...], 0.0)
    g = jnp.dot(h, w_ref[...], preferred_element_type=jnp.float32) * dinv[:, None]
    g_ref[0] = g[:, :DH]
    g_ref[1] = g[:, DH:]


def _fin_body(pd_ref, a_ref, b_ref, o_ref):
    dinv = _dinv_of(pd_ref)
    a = jnp.concatenate([a_ref[0], a_ref[1]], axis=1)
    o_ref[...] = a * dinv[:, None] + b_ref[...]


_PD_SPEC = pl.BlockSpec((2, BLK, DEG_W), lambda i: (0, i, 0))
_HALVES_SPEC = pl.BlockSpec((2, BLK, DH), lambda i: (0, i, 0))
_FULL_SPEC = pl.BlockSpec((BLK, D), lambda i: (i, 0))
_W_SPEC = pl.BlockSpec((D, D), lambda i: (0, 0))
_B_SPEC = pl.BlockSpec((1, D), lambda i: (0, 0))

_mm1 = pl.pallas_call(
    _mm1_body,
    grid=(NP // BLK,),
    in_specs=[_PD_SPEC, _FULL_SPEC, _W_SPEC],
    out_specs=_HALVES_SPEC,
    out_shape=jax.ShapeDtypeStruct((2, NP, DH), jnp.float32),
)

_mm2 = pl.pallas_call(
    _mm2_body,
    grid=(NP // BLK,),
    in_specs=[_PD_SPEC, _HALVES_SPEC, _B_SPEC, _W_SPEC],
    out_specs=_HALVES_SPEC,
    out_shape=jax.ShapeDtypeStruct((2, NP, DH), jnp.float32),
)

_fin = pl.pallas_call(
    _fin_body,
    grid=(NP // BLK,),
    in_specs=[_PD_SPEC, _HALVES_SPEC, _B_SPEC],
    out_specs=_FULL_SPEC,
    out_shape=jax.ShapeDtypeStruct((NP, D), jnp.float32),
)


@jax.jit
def _run(x, src, dst, W1, b1, W2, b2):
    dst_deg = dst.reshape(NC * NS, DEG_CHUNKS, CH)
    pad = ((0, 0), (0, AGG_EDGES_PAD - AGG_EDGES_PER_TILE))
    src_p = jnp.pad(src.reshape(NS, AGG_EDGES_PER_TILE), pad,
                    constant_values=PAD_NODE).reshape(NS, AGG_CHUNKS, ACH)
    src2 = jnp.stack([src_p, src_p + NP])
    dst_r = jnp.pad(dst.reshape(NS, AGG_EDGES_PER_TILE), pad,
                    constant_values=PAD_NODE).reshape(NS, AGG_CHUNKS, ACH)
    b1r = b1.reshape(1, D)
    b2r = b2.reshape(1, D)

    xp = jnp.pad(x, ((0, NP - N), (0, 0)))
    pd = _deg_kernel(dst_deg).reshape(NC, NP, DEG_W)
    g1 = _mm1(pd, xp, W1)
    a1 = _agg_kernel(g1.reshape(NC * NP, DH), src2, dst_r)
    g2 = _mm2(pd, a1.reshape(NC, NP, DH), b1r, W2)
    a2 = _agg_kernel(g2.reshape(NC * NP, DH), src2, dst_r)
    return _fin(pd, a2.reshape(NC, NP, DH), b2r)[:N]


def kernel(x, edge_index, W1, b1, W2, b2):
    src = edge_index[0].astype(jnp.int32)
    dst = edge_index[1].astype(jnp.int32)
    return _run(x, src, dst, W1, b1, W2, b2)


# speedup vs baseline: 1.7395x; 1.7395x over previous
import functools

import jax
import jax.numpy as jnp
from jax import lax
from jax.experimental import pallas as pl
from jax.experimental.pallas import tpu as pltpu
from jax.experimental.pallas import tpu_sc as plsc

N = 10000
NP = 10240
E = 320000
D = 128
DH = D // 2
NC = 2
NS = 16
CH = 80
ROWS_PER_TILE = NP // NS
ACH = 128
AGG_EDGES_PER_TILE = E // NS
AGG_CHUNKS = 158
AGG_EDGES_PAD = AGG_CHUNKS * ACH
PAD_NODE = NP - 8
DEG_EDGES_PER_TILE = E // (NC * NS)
DEG_CHUNKS = DEG_EDGES_PER_TILE // CH
DEG_W = 16
BLK = 1024

_MESH = plsc.VectorSubcoreMesh(core_axis_name="c", subcore_axis_name="s")


@functools.partial(
    pl.kernel,
    out_type=jax.ShapeDtypeStruct((NC * NP, DEG_W), jnp.float32),
    mesh=_MESH,
    compiler_params=pltpu.CompilerParams(use_tc_tiling_on_sc=False),
    scratch_types=[
        pltpu.VMEM((DEG_CHUNKS, CH), jnp.int32),
        pltpu.VMEM((CH, DEG_W), jnp.float32),
        pltpu.VMEM((ROWS_PER_TILE, DEG_W), jnp.float32),
        pltpu.VMEM_SHARED((NP, DEG_W), jnp.float32),
    ],
)
def _deg_kernel(dst_hbm, out_hbm, idx_v, ones_v, zeros_v, acc_sh):
    c = lax.axis_index("c")
    s = lax.axis_index("s")
    wid = c * NS + s
    one_row = jnp.ones((16,), jnp.float32)
    zero_row = jnp.zeros((16,), jnp.float32)

    @pl.loop(0, CH)
    def _(i):
        ones_v[i] = one_row

    @pl.loop(0, ROWS_PER_TILE)
    def _(i):
        zeros_v[i] = zero_row

    pltpu.sync_copy(zeros_v, acc_sh.at[pl.ds(s * ROWS_PER_TILE, ROWS_PER_TILE)])
    pltpu.sync_copy(dst_hbm.at[wid], idx_v)
    plsc.subcore_barrier()

    @pl.loop(0, DEG_CHUNKS)
    def _(j):
        pltpu.sync_copy(ones_v, acc_sh.at[idx_v.at[j]], add=True)

    plsc.subcore_barrier()
    pltpu.sync_copy(acc_sh.at[pl.ds(s * ROWS_PER_TILE, ROWS_PER_TILE)],
                    out_hbm.at[pl.ds(c * NP + s * ROWS_PER_TILE, ROWS_PER_TILE)])


@functools.partial(
    pl.kernel,
    out_type=jax.ShapeDtypeStruct((NC * NP, DH), jnp.float32),
    mesh=_MESH,
    compiler_params=pltpu.CompilerParams(use_tc_tiling_on_sc=False),
    scratch_types=[
        pltpu.VMEM((AGG_CHUNKS, ACH), jnp.int32),
        pltpu.VMEM((AGG_CHUNKS, ACH), jnp.int32),
        pltpu.VMEM((ACH, DH), jnp.float32),
        pltpu.VMEM((ACH, DH), jnp.float32),
        pltpu.VMEM_SHARED((NP, DH), jnp.float32),
        pltpu.SemaphoreType.DMA,
        pltpu.SemaphoreType.DMA,
    ],
)
def _agg_kernel(g_hbm, src2_hbm, dst_hbm, out_hbm,
                src_v, dst_v, rows0_v, rows1_v, acc_sh, sem0, sem1):
    c = lax.axis_index("c")
    s = lax.axis_index("s")
    base = c * NP + s * ROWS_PER_TILE
    pltpu.sync_copy(g_hbm.at[pl.ds(base, ROWS_PER_TILE)],
                    acc_sh.at[pl.ds(s * ROWS_PER_TILE, ROWS_PER_TILE)])
    pltpu.sync_copy(src2_hbm.at[c, s], src_v)
    pltpu.sync_copy(dst_hbm.at[s], dst_v)
    plsc.subcore_barrier()

    pltpu.async_copy(g_hbm.at[src_v.at[0]], rows0_v, sem0)
    pltpu.async_copy(g_hbm.at[src_v.at[1]], rows1_v, sem1)

    @pl.loop(0, AGG_CHUNKS, step=2)
    def _(j):
        pltpu.make_async_copy(g_hbm.at[src_v.at[j]], rows0_v, sem0).wait()
        pltpu.sync_copy(rows0_v, acc_sh.at[dst_v.at[j]], add=True)

        @pl.when(j + 2 < AGG_CHUNKS)
        def _():
            pltpu.async_copy(g_hbm.at[src_v.at[j + 2]], rows0_v, sem0)

        pltpu.make_async_copy(g_hbm.at[src_v.at[j + 1]], rows1_v, sem1).wait()
        pltpu.sync_copy(rows1_v, acc_sh.at[dst_v.at[j + 1]], add=True)

        @pl.when(j + 3 < AGG_CHUNKS)
        def _():
            pltpu.async_copy(g_hbm.at[src_v.at[j + 3]], rows1_v, sem1)

    plsc.subcore_barrier()
    pltpu.sync_copy(acc_sh.at[pl.ds(s * ROWS_PER_TILE, ROWS_PER_TILE)],
                    out_hbm.at[pl.ds(base, ROWS_PER_TILE)])


def _dinv_of(pd_ref):
    deg = pd_ref[0, :, 0] + pd_ref[1, :, 0] + 1.0
    return lax.rsqrt(deg)


def _mm1_body(pd_ref, x_ref, w_ref, g_ref):
    dinv = _dinv_of(pd_ref)
    h = jnp.dot(x_ref[...], w_ref[...], preferred_element_type=jnp.float32)
    g = h * dinv[:, None]
    g_ref[0] = g[:, :DH]
    g_ref[1] = g[:, DH:]


def _mm2_body(pd_ref, a_ref, b_ref, w_ref, g_ref):
    dinv = _dinv_of(pd_ref)
    a = jnp.concatenate([a_ref[0], a_ref[1]], axis=1)
    h = jnp.maximum(a * dinv[:, None] + b_ref[...], 0.0)
    g = jnp.dot(h, w_ref[...], preferred_element_type=jnp.float32) * dinv[:, None]
    g_ref[0] = g[:, :DH]
    g_ref[1] = g[:, DH:]


def _fin_body(pd_ref, a_ref, b_ref, o_ref):
    dinv = _dinv_of(pd_ref)
    a = jnp.concatenate([a_ref[0], a_ref[1]], axis=1)
    o_ref[...] = a * dinv[:, None] + b_ref[...]


_PD_SPEC = pl.BlockSpec((2, BLK, DEG_W), lambda i: (0, i, 0))
_HALVES_SPEC = pl.BlockSpec((2, BLK, DH), lambda i: (0, i, 0))
_FULL_SPEC = pl.BlockSpec((BLK, D), lambda i: (i, 0))
_W_SPEC = pl.BlockSpec((D, D), lambda i: (0, 0))
_B_SPEC = pl.BlockSpec((1, D), lambda i: (0, 0))

_mm1 = pl.pallas_call(
    _mm1_body,
    grid=(NP // BLK,),
    in_specs=[_PD_SPEC, _FULL_SPEC, _W_SPEC],
    out_specs=_HALVES_SPEC,
    out_shape=jax.ShapeDtypeStruct((2, NP, DH), jnp.float32),
)

_mm2 = pl.pallas_call(
    _mm2_body,
    grid=(NP // BLK,),
    in_specs=[_PD_SPEC, _HALVES_SPEC, _B_SPEC, _W_SPEC],
    out_specs=_HALVES_SPEC,
    out_shape=jax.ShapeDtypeStruct((2, NP, DH), jnp.float32),
)

_fin = pl.pallas_call(
    _fin_body,
    grid=(NP // BLK,),
    in_specs=[_PD_SPEC, _HALVES_SPEC, _B_SPEC],
    out_specs=_FULL_SPEC,
    out_shape=jax.ShapeDtypeStruct((NP, D), jnp.float32),
)


@jax.jit
def _run(x, src, dst, W1, b1, W2, b2):
    dst_deg = dst.reshape(NC * NS, DEG_CHUNKS, CH)
    padlen = AGG_EDGES_PAD - AGG_EDGES_PER_TILE
    pad_row = jnp.broadcast_to(
        N + (jnp.arange(padlen, dtype=jnp.int32) % (NP - N)), (NS, padlen))
    src_p = jnp.concatenate(
        [src.reshape(NS, AGG_EDGES_PER_TILE), pad_row],
        axis=1).reshape(NS, AGG_CHUNKS, ACH)
    src2 = jnp.stack([src_p, src_p + NP])
    dst_r = jnp.concatenate(
        [dst.reshape(NS, AGG_EDGES_PER_TILE), pad_row],
        axis=1).reshape(NS, AGG_CHUNKS, ACH)
    b1r = b1.reshape(1, D)
    b2r = b2.reshape(1, D)

    xp = jnp.pad(x, ((0, NP - N), (0, 0)))
    pd = _deg_kernel(dst_deg).reshape(NC, NP, DEG_W)
    g1 = _mm1(pd, xp, W1)
    a1 = _agg_kernel(g1.reshape(NC * NP, DH), src2, dst_r)
    g2 = _mm2(pd, a1.reshape(NC, NP, DH), b1r, W2)
    a2 = _agg_kernel(g2.reshape(NC * NP, DH), src2, dst_r)
    return _fin(pd, a2.reshape(NC, NP, DH), b2r)[:N]


def kernel(x, edge_index, W1, b1, W2, b2):
    src = edge_index[0].astype(jnp.int32)
    dst = edge_index[1].astype(jnp.int32)
    return _run(x, src, dst, W1, b1, W2, b2)


# speedup vs baseline: 2.0372x; 1.1711x over previous
import functools

import jax
import jax.numpy as jnp
from jax import lax
from jax.experimental import pallas as pl
from jax.experimental.pallas import tpu as pltpu
from jax.experimental.pallas import tpu_sc as plsc

N = 10000
NP = 10240
E = 320000
D = 128
DH = D // 2
NC = 2
NS = 16
CH = 80
ROWS_PER_TILE = NP // NS
ACH = 128
AGG_EDGES_PER_TILE = E // NS
AGG_CHUNKS = 160
AGG_EDGES_PAD = AGG_CHUNKS * ACH
PAD_NODE = NP - 8
DEG_EDGES_PER_TILE = E // (NC * NS)
DEG_CHUNKS = DEG_EDGES_PER_TILE // CH
DEG_W = 16
BLK = 1024

_MESH = plsc.VectorSubcoreMesh(core_axis_name="c", subcore_axis_name="s")


@functools.partial(
    pl.kernel,
    out_type=jax.ShapeDtypeStruct((NC * NP, DEG_W), jnp.float32),
    mesh=_MESH,
    compiler_params=pltpu.CompilerParams(use_tc_tiling_on_sc=False),
    scratch_types=[
        pltpu.VMEM((DEG_CHUNKS, CH), jnp.int32),
        pltpu.VMEM((CH, DEG_W), jnp.float32),
        pltpu.VMEM((ROWS_PER_TILE, DEG_W), jnp.float32),
        pltpu.VMEM_SHARED((NP, DEG_W), jnp.float32),
    ],
)
def _deg_kernel(dst_hbm, out_hbm, idx_v, ones_v, zeros_v, acc_sh):
    c = lax.axis_index("c")
    s = lax.axis_index("s")
    wid = c * NS + s
    one_row = jnp.ones((16,), jnp.float32)
    zero_row = jnp.zeros((16,), jnp.float32)

    @pl.loop(0, CH)
    def _(i):
        ones_v[i] = one_row

    @pl.loop(0, ROWS_PER_TILE)
    def _(i):
        zeros_v[i] = zero_row

    pltpu.sync_copy(zeros_v, acc_sh.at[pl.ds(s * ROWS_PER_TILE, ROWS_PER_TILE)])
    pltpu.sync_copy(dst_hbm.at[wid], idx_v)
    plsc.subcore_barrier()

    @pl.loop(0, DEG_CHUNKS)
    def _(j):
        pltpu.sync_copy(ones_v, acc_sh.at[idx_v.at[j]], add=True)

    plsc.subcore_barrier()
    pltpu.sync_copy(acc_sh.at[pl.ds(s * ROWS_PER_TILE, ROWS_PER_TILE)],
                    out_hbm.at[pl.ds(c * NP + s * ROWS_PER_TILE, ROWS_PER_TILE)])


@functools.partial(
    pl.kernel,
    out_type=jax.ShapeDtypeStruct((NC * NP, DH), jnp.float32),
    mesh=_MESH,
    compiler_params=pltpu.CompilerParams(use_tc_tiling_on_sc=False),
    scratch_types=[
        pltpu.VMEM((AGG_CHUNKS, ACH), jnp.int32),
        pltpu.VMEM((AGG_CHUNKS, ACH), jnp.int32),
        pltpu.VMEM((ACH, DH), jnp.float32),
        pltpu.VMEM((ACH, DH), jnp.float32),
        pltpu.VMEM((ACH, DH), jnp.float32),
        pltpu.VMEM((ACH, DH), jnp.float32),
        pltpu.VMEM_SHARED((NP, DH), jnp.float32),
        pltpu.SemaphoreType.DMA,
        pltpu.SemaphoreType.DMA,
        pltpu.SemaphoreType.DMA,
        pltpu.SemaphoreType.DMA,
    ],
)
def _agg_kernel(g_hbm, src2_hbm, dst_hbm, out_hbm,
                src_v, dst_v, rows0_v, rows1_v, rows2_v, rows3_v, acc_sh,
                sem0, sem1, sem2, sem3):
    c = lax.axis_index("c")
    s = lax.axis_index("s")
    base = c * NP + s * ROWS_PER_TILE
    pltpu.sync_copy(g_hbm.at[pl.ds(base, ROWS_PER_TILE)],
                    acc_sh.at[pl.ds(s * ROWS_PER_TILE, ROWS_PER_TILE)])
    pltpu.sync_copy(src2_hbm.at[c, s], src_v)
    pltpu.sync_copy(dst_hbm.at[s], dst_v)
    plsc.subcore_barrier()

    bufs = (rows0_v, rows1_v, rows2_v, rows3_v)
    sems = (sem0, sem1, sem2, sem3)
    for b in range(4):
        pltpu.async_copy(g_hbm.at[src_v.at[b]], bufs[b], sems[b])

    @pl.loop(0, AGG_CHUNKS, step=4)
    def _(j):
        for b in range(4):
            pltpu.make_async_copy(
                g_hbm.at[src_v.at[j + b]], bufs[b], sems[b]).wait()
            pltpu.sync_copy(bufs[b], acc_sh.at[dst_v.at[j + b]], add=True)

            @pl.when(j + b + 4 < AGG_CHUNKS)
            def _():
                pltpu.async_copy(g_hbm.at[src_v.at[j + b + 4]], bufs[b], sems[b])

    plsc.subcore_barrier()
    pltpu.sync_copy(acc_sh.at[pl.ds(s * ROWS_PER_TILE, ROWS_PER_TILE)],
                    out_hbm.at[pl.ds(base, ROWS_PER_TILE)])


def _dinv_of(pd_ref):
    deg = pd_ref[0, :, 0] + pd_ref[1, :, 0] + 1.0
    return lax.rsqrt(deg)


def _mm1_body(pd_ref, x_ref, w_ref, g_ref):
    dinv = _dinv_of(pd_ref)
    h = jnp.dot(x_ref[...], w_ref[...], preferred_element_type=jnp.float32)
    g = h * dinv[:, None]
    g_ref[0] = g[:, :DH]
    g_ref[1] = g[:, DH:]


def _mm2_body(pd_ref, a_ref, b_ref, w_ref, g_ref):
    dinv = _dinv_of(pd_ref)
    a = jnp.concatenate([a_ref[0], a_ref[1]], axis=1)
    h = jnp.maximum(a * dinv[:, None] + b_ref[...], 0.0)
    g = jnp.dot(h, w_ref[...], preferred_element_type=jnp.float32) * dinv[:, None]
    g_ref[0] = g[:, :DH]
    g_ref[1] = g[:, DH:]


def _fin_body(pd_ref, a_ref, b_ref, o_ref):
    dinv = _dinv_of(pd_ref)
    a = jnp.concatenate([a_ref[0], a_ref[1]], axis=1)
    o_ref[...] = a * dinv[:, None] + b_ref[...]


_PD_SPEC = pl.BlockSpec((2, BLK, DEG_W), lambda i: (0, i, 0))
_HALVES_SPEC = pl.BlockSpec((2, BLK, DH), lambda i: (0, i, 0))
_FULL_SPEC = pl.BlockSpec((BLK, D), lambda i: (i, 0))
_W_SPEC = pl.BlockSpec((D, D), lambda i: (0, 0))
_B_SPEC = pl.BlockSpec((1, D), lambda i: (0, 0))

_mm1 = pl.pallas_call(
    _mm1_body,
    grid=(NP // BLK,),
    in_specs=[_PD_SPEC, _FULL_SPEC, _W_SPEC],
    out_specs=_HALVES_SPEC,
    out_shape=jax.ShapeDtypeStruct((2, NP, DH), jnp.float32),
)

_mm2 = pl.pallas_call(
    _mm2_body,
    grid=(NP // BLK,),
    in_specs=[_PD_SPEC, _HALVES_SPEC, _B_SPEC, _W_SPEC],
    out_specs=_HALVES_SPEC,
    out_shape=jax.ShapeDtypeStruct((2, NP, DH), jnp.float32),
)

_fin = pl.pallas_call(
    _fin_body,
    grid=(NP // BLK,),
    in_specs=[_PD_SPEC, _HALVES_SPEC, _B_SPEC],
    out_specs=_FULL_SPEC,
    out_shape=jax.ShapeDtypeStruct((NP, D), jnp.float32),
)


@jax.jit
def _run(x, src, dst, W1, b1, W2, b2):
    dst_deg = dst.reshape(NC * NS, DEG_CHUNKS, CH)
    padlen = AGG_EDGES_PAD - AGG_EDGES_PER_TILE
    pad_row = jnp.broadcast_to(
        N + (jnp.arange(padlen, dtype=jnp.int32) % (NP - N)), (NS, padlen))
    src_p = jnp.concatenate(
        [src.reshape(NS, AGG_EDGES_PER_TILE), pad_row],
        axis=1).reshape(NS, AGG_CHUNKS, ACH)
    src2 = jnp.stack([src_p, src_p + NP])
    dst_r = jnp.concatenate(
        [dst.reshape(NS, AGG_EDGES_PER_TILE), pad_row],
        axis=1).reshape(NS, AGG_CHUNKS, ACH)
    b1r = b1.reshape(1, D)
    b2r = b2.reshape(1, D)

    xp = jnp.pad(x, ((0, NP - N), (0, 0)))
    pd = _deg_kernel(dst_deg).reshape(NC, NP, DEG_W)
    g1 = _mm1(pd, xp, W1)
    a1 = _agg_kernel(g1.reshape(NC * NP, DH), src2, dst_r)
    g2 = _mm2(pd, a1.reshape(NC, NP, DH), b1r, W2)
    a2 = _agg_kernel(g2.reshape(NC * NP, DH), src2, dst_r)
    return _fin(pd, a2.reshape(NC, NP, DH), b2r)[:N]


def kernel(x, edge_index, W1, b1, W2, b2):
    src = edge_index[0].astype(jnp.int32)
    dst = edge_index[1].astype(jnp.int32)
    return _run(x, src, dst, W1, b1, W2, b2)
